# baseline (speedup 1.0000x reference)
import functools

import jax
import jax.numpy as jnp
from jax import lax
from jax.experimental import pallas as pl
from jax.experimental.pallas import tpu as pltpu
from jax.experimental.pallas import tpu_sc as plsc

S, E, C = 4096, 16, 512
NW = 32
TPW = S // NW
L = 16
NJ = C // L


def _sc_body(g1_hbm, l1_hbm, g2_hbm, l2_hbm, m1_hbm, m2_hbm, out_hbm,
             g1v, g2v, m1v, m2v, l1v, l2v, outv, s1, s2, so):
    wid = lax.axis_index("s") * 2 + lax.axis_index("c")
    base = wid * TPW
    pltpu.sync_copy(g1_hbm.at[pl.ds(base, TPW)], g1v.at[pl.ds(0, TPW)])
    pltpu.sync_copy(g2_hbm.at[pl.ds(base, TPW)], g2v.at[pl.ds(0, TPW)])
    pltpu.sync_copy(m1_hbm.at[pl.ds(base, TPW)], m1v)
    pltpu.sync_copy(m2_hbm.at[pl.ds(base, TPW)], m2v)

    def in_copies(tok, slot):
        return (
            pltpu.make_async_copy(l1_hbm.at[base + tok], l1v.at[slot], s1.at[slot]),
            pltpu.make_async_copy(l2_hbm.at[base + tok], l2v.at[slot], s2.at[slot]),
        )

    def out_copy(tok, slot):
        return pltpu.make_async_copy(outv.at[slot], out_hbm.at[base + tok], so.at[slot])

    for cp in in_copies(0, 0):
        cp.start()

    def token(i, carry):
        slot = lax.rem(i, 2)
        nslot = lax.rem(i + 1, 2)

        @pl.when(i + 1 < TPW)
        def _():
            for cp in in_copies(i + 1, nslot):
                cp.start()

        for cp in in_copies(i, slot):
            cp.wait()

        @pl.when(i >= 2)
        def _():
            out_copy(i - 2, slot).wait()

        g1 = g1v[pl.ds(i, L)][0]
        g2 = g2v[pl.ds(i, L)][0]
        am = g1 * m1v[i]
        bm = g2 * m2v[i]
        av = [am[e] for e in range(E)]
        bv = [bm[e] for e in range(E)]
        for j in range(NJ):
            l1j = l1v[slot, pl.ds(j * L, L)]
            l2j = l2v[slot, pl.ds(j * L, L)]
            for e in range(E):
                outv[slot, e, pl.ds(j * L, L)] = av[e] * l1j + bv[e] * l2j

        out_copy(i, slot).start()
        return carry

    lax.fori_loop(0, TPW, token, 0)
    out_copy(TPW - 2, lax.rem(jnp.int32(TPW - 2), 2)).wait()
    out_copy(TPW - 1, lax.rem(jnp.int32(TPW - 1), 2)).wait()


def kernel(gates1_s, locations1_sc, gates2_s, locations2_sc, mask1_float, mask2_float):
    mesh = plsc.VectorSubcoreMesh(core_axis_name="c", subcore_axis_name="s")
    k = functools.partial(
        pl.kernel,
        out_type=jax.ShapeDtypeStruct((S, E, C), jnp.float32),
        mesh=mesh,
        scratch_types=[
            pltpu.VMEM((TPW + L,), jnp.float32),
            pltpu.VMEM((TPW + L,), jnp.float32),
            pltpu.VMEM((TPW, E), jnp.float32),
            pltpu.VMEM((TPW, E), jnp.float32),
            pltpu.VMEM((2, C), jnp.float32),
            pltpu.VMEM((2, C), jnp.float32),
            pltpu.VMEM((2, E, C), jnp.float32),
            pltpu.SemaphoreType.DMA((2,)),
            pltpu.SemaphoreType.DMA((2,)),
            pltpu.SemaphoreType.DMA((2,)),
        ],
    )(_sc_body)
    return k(gates1_s, locations1_sc, gates2_s, locations2_sc,
             mask1_float, mask2_float)


# speedup vs baseline: 1.8909x; 1.8909x over previous
import jax
import jax.numpy as jnp
from jax import lax
from jax.experimental import pallas as pl
from jax.experimental.pallas import tpu as pltpu

S, E, C = 4096, 16, 512
CT = 1024
QT = 128
CH = 4
NS = 6
NQ = CT // QT
NSTEPS = S // CT
NCHUNKS = S // QT


def _body(g1_ref, l1_ref, g2_ref, l2_ref, m1_ref, m2_ref, o_hbm, obuf, sems):
    i = pl.program_id(0)
    g1m1 = (g1_ref[...] * m1_ref[...])[:, :, None]
    g2m2 = (g2_ref[...] * m2_ref[...])[:, :, None]

    for q in range(NQ):
        k = i * NQ + q
        slot = lax.rem(k, NS)

        @pl.when(k >= NS)
        def _():
            pltpu.make_async_copy(
                obuf.at[slot], o_hbm.at[pl.ds((k - NS) * QT, QT)], sems.at[slot]
            ).wait()

        for b in range(0, QT, CH):
            src = slice(q * QT + b, q * QT + b + CH)
            dst = slice(b, b + CH)
            l1 = l1_ref[src][:, None, :]
            l2 = l2_ref[src][:, None, :]
            obuf[slot, dst] = g1m1[src] * l1 + g2m2[src] * l2

        pltpu.make_async_copy(
            obuf.at[slot], o_hbm.at[pl.ds(k * QT, QT)], sems.at[slot]
        ).start()

    @pl.when(i == NSTEPS - 1)
    def _():
        for kk in range(NCHUNKS - NS, NCHUNKS):
            pltpu.make_async_copy(
                obuf.at[kk % NS], o_hbm.at[pl.ds(kk * QT, QT)], sems.at[kk % NS]
            ).wait()


def kernel(gates1_s, locations1_sc, gates2_s, locations2_sc, mask1_float, mask2_float):
    g1 = gates1_s.reshape(S, 1)
    g2 = gates2_s.reshape(S, 1)
    return pl.pallas_call(
        _body,
        grid=(NSTEPS,),
        in_specs=[
            pl.BlockSpec((CT, 1), lambda i: (i, 0)),
            pl.BlockSpec((CT, C), lambda i: (i, 0)),
            pl.BlockSpec((CT, 1), lambda i: (i, 0)),
            pl.BlockSpec((CT, C), lambda i: (i, 0)),
            pl.BlockSpec((CT, E), lambda i: (i, 0)),
            pl.BlockSpec((CT, E), lambda i: (i, 0)),
        ],
        out_specs=pl.BlockSpec(memory_space=pl.ANY),
        out_shape=jax.ShapeDtypeStruct((S, E, C), jnp.float32),
        scratch_shapes=[
            pltpu.VMEM((NS, QT, E, C), jnp.float32),
            pltpu.SemaphoreType.DMA((NS,)),
        ],
    )(g1, locations1_sc, g2, locations2_sc, mask1_float, mask2_float)


# speedup vs baseline: 1.9367x; 1.0242x over previous
import jax
import jax.numpy as jnp
from jax import lax
from jax.experimental import pallas as pl
from jax.experimental.pallas import tpu as pltpu

S, E, C = 4096, 16, 512
CT = 512
QT = 128
CH = 4
NS = 8
NQ = CT // QT
NSTEPS = S // CT
NCHUNKS = S // QT


def _body(g1_ref, l1_ref, g2_ref, l2_ref, m1_ref, m2_ref, o_hbm, obuf, sems):
    i = pl.program_id(0)
    g1m1 = (g1_ref[...] * m1_ref[...])[:, :, None]
    g2m2 = (g2_ref[...] * m2_ref[...])[:, :, None]

    for q in range(NQ):
        k = i * NQ + q
        slot = lax.rem(k, NS)

        @pl.when(k >= NS)
        def _():
            pltpu.make_async_copy(
                obuf.at[slot], o_hbm.at[pl.ds((k - NS) * QT, QT)], sems.at[slot]
            ).wait()

        for b in range(0, QT, CH):
            src = slice(q * QT + b, q * QT + b + CH)
            dst = slice(b, b + CH)
            l1 = l1_ref[src][:, None, :]
            l2 = l2_ref[src][:, None, :]
            obuf[slot, dst] = g1m1[src] * l1 + g2m2[src] * l2

        pltpu.make_async_copy(
            obuf.at[slot], o_hbm.at[pl.ds(k * QT, QT)], sems.at[slot]
        ).start()

    @pl.when(i == NSTEPS - 1)
    def _():
        for kk in range(NCHUNKS - NS, NCHUNKS):
            pltpu.make_async_copy(
                obuf.at[kk % NS], o_hbm.at[pl.ds(kk * QT, QT)], sems.at[kk % NS]
            ).wait()


def kernel(gates1_s, locations1_sc, gates2_s, locations2_sc, mask1_float, mask2_float):
    g1 = gates1_s.reshape(S, 1)
    g2 = gates2_s.reshape(S, 1)
    return pl.pallas_call(
        _body,
        grid=(NSTEPS,),
        in_specs=[
            pl.BlockSpec((CT, 1), lambda i: (i, 0)),
            pl.BlockSpec((CT, C), lambda i: (i, 0)),
            pl.BlockSpec((CT, 1), lambda i: (i, 0)),
            pl.BlockSpec((CT, C), lambda i: (i, 0)),
            pl.BlockSpec((CT, E), lambda i: (i, 0)),
            pl.BlockSpec((CT, E), lambda i: (i, 0)),
        ],
        out_specs=pl.BlockSpec(memory_space=pl.ANY),
        out_shape=jax.ShapeDtypeStruct((S, E, C), jnp.float32),
        scratch_shapes=[
            pltpu.VMEM((NS, QT, E, C), jnp.float32),
            pltpu.SemaphoreType.DMA((NS,)),
        ],
    )(g1, locations1_sc, g2, locations2_sc, mask1_float, mask2_float)
